# Initial kernel scaffold; baseline (speedup 1.0000x reference)
import jax
import jax.numpy as jnp
from jax.experimental import pallas as pl


def kernel(bond_vectors, edge_index):
    raise NotImplementedError("write your pallas kernel here")


# speedup vs baseline: 10.1553x; 10.1553x over previous
import functools

import jax
import jax.numpy as jnp
import numpy as np
from jax import lax
from jax.experimental import pallas as pl
from jax.experimental.pallas import tpu as pltpu
from jax.experimental.pallas import tpu_sc as plsc

N_NODES = 100000
N_EDGES = 6400000
E0 = np.float32(4.0 * ((1.0 / 3.0) ** 12 - (1.0 / 3.0) ** 6))

NG = 33792
ACCW = NG * 3
NSC = 2
EDGES_PER_SC = N_EDGES // NSC
CHUNK = 3200
NCHUNKS = EDGES_PER_SC // CHUNK
STEPS = CHUNK // 16


def _sc_body(bv_hbm, dst_hbm, p_out, e_out, acc, bvb, dstb, evec):
    c = lax.axis_index("c")
    s = lax.axis_index("s")
    g = jnp.where(s < 6, 0, jnp.where(s < 11, 1, 2))
    slot = s - jnp.where(s < 6, 0, jnp.where(s < 11, 6, 11))
    nslots = jnp.where(s < 6, 6, 5)
    lo = g * NG
    iota = lax.iota(jnp.int32, 16)

    zero16 = jnp.zeros((16,), jnp.float32)

    def zbody(i, carry):
        acc[pl.ds(i * 16, 16)] = zero16
        return carry

    lax.fori_loop(0, ACCW // 16, zbody, 0)

    count = (NCHUNKS - slot + nslots - 1) // nslots

    def chunk_body(j, ecarry):
        ci = slot + j * nslots
        ebase = c * EDGES_PER_SC + ci * CHUNK
        pltpu.sync_copy(bv_hbm.at[pl.ds(ebase * 3, CHUNK * 3)], bvb)
        pltpu.sync_copy(dst_hbm.at[pl.ds(ebase, CHUNK)], dstb)

        def step(i, ecar):
            idx = i * 48 + iota * 3
            x = plsc.load_gather(bvb, [idx])
            y = plsc.load_gather(bvb, [idx + 1])
            z = plsc.load_gather(bvb, [idx + 2])
            d = dstb[pl.ds(i * 16, 16)]
            r2 = x * x + y * y + z * z
            inv = 1.0 / r2
            c6 = inv * inv * inv
            c12 = c6 * c6
            coef = (2.0 * c12 - c6) * inv
            lrow = d - lo
            m = (lrow >= 0) & (lrow < NG)
            a0 = jnp.where(m, lrow * 3, 0)
            plsc.addupdate_scatter(acc, [a0], coef * x, mask=m)
            plsc.addupdate_scatter(acc, [a0 + 1], coef * y, mask=m)
            plsc.addupdate_scatter(acc, [a0 + 2], coef * z, mask=m)
            return ecar + (c12 - c6)

        return lax.fori_loop(0, STEPS, step, ecarry)

    e16 = lax.fori_loop(0, count, chunk_body, jnp.zeros((16,), jnp.float32))
    evec[...] = e16 * 4.0
    pltpu.sync_copy(acc, p_out.at[c, s])
    pltpu.sync_copy(evec, e_out.at[c, s])


def _combine_body(q_ref, e_ref, f12_ref, fm24_ref, pe_ref):
    i = pl.program_id(0)
    q = q_ref[...]
    s0 = jnp.sum(q[0:6], axis=0) + jnp.sum(q[16:22], axis=0)
    s1 = jnp.sum(q[6:11], axis=0) + jnp.sum(q[22:27], axis=0)
    s2 = jnp.sum(q[11:16], axis=0) + jnp.sum(q[27:32], axis=0)
    f = jnp.stack([s0, s1, s2]) * 12.0
    f12_ref[...] = f
    fm24_ref[...] = f * -2.0

    @pl.when(i == 0)
    def _():
        pe = 0.5 * (jnp.sum(e_ref[...]) * (1.0 / 3.0) - E0 * np.float32(N_EDGES))
        pe_ref[...] = jnp.full((1, 1), pe, jnp.float32)


def kernel(bond_vectors, edge_index):
    bv_flat = bond_vectors.reshape(-1)
    dst = edge_index[1]

    mesh = plsc.VectorSubcoreMesh(
        core_axis_name="c", subcore_axis_name="s", num_cores=2, num_subcores=16
    )
    sc_kernel = pl.kernel(
        _sc_body,
        out_type=(
            jax.ShapeDtypeStruct((2, 16, ACCW), jnp.float32),
            jax.ShapeDtypeStruct((2, 16, 16), jnp.float32),
        ),
        mesh=mesh,
        compiler_params=pltpu.CompilerParams(needs_layout_passes=False),
        scratch_types=[
            pltpu.VMEM((ACCW,), jnp.float32),
            pltpu.VMEM((CHUNK * 3,), jnp.float32),
            pltpu.VMEM((CHUNK,), jnp.int32),
            pltpu.VMEM((16,), jnp.float32),
        ],
    )
    p_parts, e_parts = sc_kernel(bv_flat, dst)

    q = p_parts.reshape(32, ACCW)
    e2 = e_parts.reshape(1, 512)
    nblk = ACCW // 512
    f12, fm24, pe = pl.pallas_call(
        _combine_body,
        grid=(nblk,),
        in_specs=[
            pl.BlockSpec((32, 512), lambda i: (0, i)),
            pl.BlockSpec((1, 512), lambda i: (0, 0)),
        ],
        out_specs=[
            pl.BlockSpec((3, 512), lambda i: (0, i)),
            pl.BlockSpec((3, 512), lambda i: (0, i)),
            pl.BlockSpec((1, 1), lambda i: (0, 0)),
        ],
        out_shape=[
            jax.ShapeDtypeStruct((3, ACCW), jnp.float32),
            jax.ShapeDtypeStruct((3, ACCW), jnp.float32),
            jax.ShapeDtypeStruct((1, 1), jnp.float32),
        ],
    )(q, e2)

    forces = f12.reshape(-1)[: N_NODES * 3].reshape(N_NODES, 3)
    analytic = fm24.reshape(-1)[: N_NODES * 3].reshape(N_NODES, 3)
    return (pe[0, 0], forces, analytic)
